# baseline (speedup 1.0000x reference)
import functools

import jax
import jax.numpy as jnp
from jax import lax
from jax.experimental import pallas as pl
from jax.experimental.pallas import tpu as pltpu
from jax.experimental.pallas import tpu_sc as plsc

TILE = 512
_NW = 32


def _meta_body(nroute, x_ref, wr_ref, bias_ref, scores_ref, dest_ref, te_ref):
    logits = jnp.dot(x_ref[...], wr_ref[...]) * bias_ref[...]
    p = jax.nn.sigmoid(logits)
    t, e = p.shape
    ii = lax.broadcasted_iota(jnp.int32, (t, e), 1)
    m1 = jnp.max(p, axis=1, keepdims=True)
    first1 = jnp.min(jnp.where(p == m1, ii, e), axis=1, keepdims=True)
    sel1 = ii == first1
    p2 = jnp.where(sel1, -1.0, p)
    m2 = jnp.max(p2, axis=1, keepdims=True)
    first2 = jnp.min(jnp.where(p2 == m2, ii, e), axis=1, keepdims=True)
    sel2 = ii == first2
    denom = m1 + m2
    scores_ref[...] = jnp.concatenate([m1 / denom, m2 / denom], axis=1)

    oh = jnp.concatenate([sel1, sel2], axis=0).astype(jnp.float32)
    pairs = 2 * t
    nblk = pairs // 128
    oh3 = oh.reshape(nblk, 128, e)
    r_i = lax.broadcasted_iota(jnp.int32, (128, 128), 0)
    c_i = lax.broadcasted_iota(jnp.int32, (128, 128), 1)
    t128 = (r_i >= c_i).astype(jnp.bfloat16)
    t128b = jnp.broadcast_to(t128[None], (nblk, 128, 128))
    within = lax.dot_general(
        t128b, oh3.astype(jnp.bfloat16), (((2,), (1,)), ((0,), (0,))),
        preferred_element_type=jnp.float32)
    sums = within[:, 127, :]
    rb = lax.broadcasted_iota(jnp.int32, (nblk, nblk), 0)
    cb = lax.broadcasted_iota(jnp.int32, (nblk, nblk), 1)
    tnb = (rb > cb).astype(jnp.bfloat16)
    offs = jnp.dot(tnb, sums.astype(jnp.bfloat16),
                   preferred_element_type=jnp.float32)

    counts = jnp.sum(oh, axis=0, keepdims=True)
    pc = jnp.ceil(counts / TILE) * TILE
    fe_r = lax.broadcasted_iota(jnp.int32, (e, e), 0)
    fe_c = lax.broadcasted_iota(jnp.int32, (e, e), 1)
    upper = (fe_r < fe_c).astype(jnp.bfloat16)
    po = jnp.dot(pc.astype(jnp.bfloat16), upper,
                 preferred_element_type=jnp.float32)

    rank_excl = within - oh3 + offs[:, None, :]
    dest3 = jnp.sum(oh3 * (rank_excl + po.reshape(1, 1, e)),
                    axis=2)
    dest_ref[...] = dest3.astype(jnp.int32)

    ends_t = jnp.transpose(po + pc)
    ntl = te_ref.shape[1]
    ti_iota = lax.broadcasted_iota(jnp.int32, (1, ntl), 1)
    tile_start = (ti_iota * TILE).astype(jnp.float32)
    raw = jnp.sum((ends_t <= tile_start).astype(jnp.int32), axis=0,
                  keepdims=True)
    del nroute
    te_ref[...] = raw


@functools.partial(jax.jit, static_argnums=(2,))
def _dispatch(xf, dest32, slots):
    t, h = xf.shape
    rows_per = t // _NW
    mesh = plsc.VectorSubcoreMesh(core_axis_name="c", subcore_axis_name="s")

    @functools.partial(
        pl.kernel, mesh=mesh,
        out_type=jax.ShapeDtypeStruct((slots, h), jnp.float32),
        scratch_types=[
            pltpu.VMEM((rows_per,), jnp.int32),
            pltpu.VMEM((rows_per,), jnp.int32),
            pltpu.VMEM((rows_per, h), jnp.float32),
            pltpu.SemaphoreType.DMA,
            pltpu.SemaphoreType.DMA,
        ],
    )
    def body(x_hbm, dest_hbm, xg_hbm, i0_v, i1_v, rows_v, sem0, sem1):
        wid = lax.axis_index("s") * 2 + lax.axis_index("c")
        base = wid * rows_per
        pltpu.sync_copy(
            dest_hbm.at[base // 128, pl.ds(lax.rem(base, 128), rows_per)],
            i0_v)
        pltpu.sync_copy(
            dest_hbm.at[(t + base) // 128,
                        pl.ds(lax.rem(t + base, 128), rows_per)],
            i1_v)
        pltpu.sync_copy(x_hbm.at[pl.ds(base, rows_per), :], rows_v)
        c0 = pltpu.async_copy(rows_v, xg_hbm.at[i0_v], sem0)
        c1 = pltpu.async_copy(rows_v, xg_hbm.at[i1_v], sem1)
        c0.wait()
        c1.wait()

    return body(xf, dest32)


def _gather2(outbuf, dest32, t):
    slots, h = outbuf.shape
    rows_per = t // _NW
    mesh = plsc.VectorSubcoreMesh(core_axis_name="c", subcore_axis_name="s")

    @functools.partial(
        pl.kernel, mesh=mesh,
        out_type=[jax.ShapeDtypeStruct((t, h), jnp.float32),
                  jax.ShapeDtypeStruct((t, h), jnp.float32)],
        scratch_types=[
            pltpu.VMEM((rows_per,), jnp.int32),
            pltpu.VMEM((rows_per,), jnp.int32),
            pltpu.VMEM((rows_per, h), jnp.float32),
            pltpu.VMEM((rows_per, h), jnp.float32),
            pltpu.SemaphoreType.DMA,
            pltpu.SemaphoreType.DMA,
        ],
    )
    def body(ob_hbm, dest_hbm, g0_hbm, g1_hbm,
             i0_v, i1_v, r0_v, r1_v, sem0, sem1):
        wid = lax.axis_index("s") * 2 + lax.axis_index("c")
        base = wid * rows_per
        pltpu.sync_copy(
            dest_hbm.at[base // 128, pl.ds(lax.rem(base, 128), rows_per)],
            i0_v)
        pltpu.sync_copy(
            dest_hbm.at[(t + base) // 128,
                        pl.ds(lax.rem(t + base, 128), rows_per)],
            i1_v)
        c0 = pltpu.async_copy(ob_hbm.at[i0_v], r0_v, sem0)
        c1 = pltpu.async_copy(ob_hbm.at[i1_v], r1_v, sem1)
        c0.wait()
        pltpu.sync_copy(r0_v, g0_hbm.at[pl.ds(base, rows_per), :])
        c1.wait()
        pltpu.sync_copy(r1_v, g1_hbm.at[pl.ds(base, rows_per), :])

    return body(outbuf, dest32)


def _expert_body(te_ref, xg_ref, wg_ref, wu_ref, wd_ref, out_ref):
    ti = pl.program_id(0)

    @pl.when(te_ref[ti] < 8)
    def _routed():
        xb = xg_ref[...].astype(jnp.bfloat16)
        wg = wg_ref[0].astype(jnp.bfloat16)
        wu = wu_ref[0].astype(jnp.bfloat16)
        wd = wd_ref[0].astype(jnp.bfloat16)
        g = jnp.dot(xb, wg, preferred_element_type=jnp.float32)
        u = jnp.dot(xb, wu, preferred_element_type=jnp.float32)
        h = ((g * jax.nn.sigmoid(g)) * u).astype(jnp.bfloat16)
        out_ref[...] = jnp.dot(h, wd, preferred_element_type=jnp.float32)


def _shared_body(x_ref, wgs_ref, wus_ref, wds_ref, out_ref):
    ic = pl.program_id(0)
    xb = x_ref[...].astype(jnp.bfloat16)
    g = jnp.dot(xb, wgs_ref[...].astype(jnp.bfloat16),
                preferred_element_type=jnp.float32)
    u = jnp.dot(xb, wus_ref[...].astype(jnp.bfloat16),
                preferred_element_type=jnp.float32)
    h = ((g * jax.nn.sigmoid(g)) * u).astype(jnp.bfloat16)
    part = jnp.dot(h, wds_ref[...].astype(jnp.bfloat16),
                   preferred_element_type=jnp.float32)

    @pl.when(ic == 0)
    def _first():
        out_ref[...] = part

    @pl.when(ic > 0)
    def _rest():
        out_ref[...] += part


def _combine_body(sha_ref, shb_ref, g0_ref, g1_ref, sc_ref, out_ref):
    s = sc_ref[...]
    out_ref[...] = (sha_ref[...] + shb_ref[...] + s[:, 0:1] * g0_ref[...]
                    + s[:, 1:2] * g1_ref[...])


def kernel(x, W_router, routing_bias, Wg_s, Wu_s, Wd_s, Wg, Wu, Wd):
    b, s_, h = x.shape
    t = b * s_
    e = Wg.shape[0]
    i = Wg.shape[2]
    xf = x.reshape(t, h)
    ntiles = -((-2 * t) // TILE) + e
    slots = ntiles * TILE
    nb = (2 * t) // 128
    ntl = max(32, ntiles)

    scores, dest32, te = pl.pallas_call(
        functools.partial(_meta_body, ntiles),
        out_shape=[
            jax.ShapeDtypeStruct((t, 2), jnp.float32),
            jax.ShapeDtypeStruct((nb, 128), jnp.int32),
            jax.ShapeDtypeStruct((1, ntl), jnp.int32),
        ],
    )(xf, W_router, routing_bias.reshape(1, e))

    te1 = te.reshape(ntl)

    n_ic = 3 if i % 3 == 0 else 1
    iblk = i // n_ic

    xg = _dispatch(xf, dest32, slots)

    shared_a = pl.pallas_call(
        _shared_body,
        grid=(1,),
        in_specs=[
            pl.BlockSpec((t, h), lambda ic: (0, 0)),
            pl.BlockSpec((h, iblk), lambda ic: (0, 0)),
            pl.BlockSpec((h, iblk), lambda ic: (0, 0)),
            pl.BlockSpec((iblk, h), lambda ic: (0, 0)),
        ],
        out_specs=pl.BlockSpec((t, h), lambda ic: (0, 0)),
        out_shape=jax.ShapeDtypeStruct((t, h), jnp.float32),
        compiler_params=pltpu.CompilerParams(
            dimension_semantics=("arbitrary",)),
    )(xf, Wg_s, Wu_s, Wd_s)
    shared_b = pl.pallas_call(
        _shared_body,
        grid=(n_ic - 1,) if n_ic > 1 else (1,),
        in_specs=[
            pl.BlockSpec((t, h), lambda ic: (0, 0)),
            pl.BlockSpec((h, iblk),
                         lambda ic: (0, ic + 1 if n_ic > 1 else 0)),
            pl.BlockSpec((h, iblk),
                         lambda ic: (0, ic + 1 if n_ic > 1 else 0)),
            pl.BlockSpec((iblk, h),
                         lambda ic: (ic + 1 if n_ic > 1 else 0, 0)),
        ],
        out_specs=pl.BlockSpec((t, h), lambda ic: (0, 0)),
        out_shape=jax.ShapeDtypeStruct((t, h), jnp.float32),
        compiler_params=pltpu.CompilerParams(
            dimension_semantics=("arbitrary",)),
    )(xf, Wg_s, Wu_s, Wd_s)

    grid_spec = pltpu.PrefetchScalarGridSpec(
        num_scalar_prefetch=1,
        grid=(ntiles,),
        in_specs=[
            pl.BlockSpec((TILE, h), lambda ti, te_r: (ti, 0)),
            pl.BlockSpec((1, h, i),
                         lambda ti, te_r: (jnp.minimum(te_r[ti], 7), 0, 0)),
            pl.BlockSpec((1, h, i),
                         lambda ti, te_r: (jnp.minimum(te_r[ti], 7), 0, 0)),
            pl.BlockSpec((1, i, h),
                         lambda ti, te_r: (jnp.minimum(te_r[ti], 7), 0, 0)),
        ],
        out_specs=pl.BlockSpec((TILE, h), lambda ti, te_r: (ti, 0)),
    )
    outbuf = pl.pallas_call(
        _expert_body,
        grid_spec=grid_spec,
        out_shape=jax.ShapeDtypeStruct((slots, h), jnp.float32),
        compiler_params=pltpu.CompilerParams(
            dimension_semantics=("arbitrary",)),
    )(te1, xg, Wg, Wu, Wd)

    g0, g1 = _gather2(outbuf, dest32, t)

    tb = t // 4
    out = pl.pallas_call(
        _combine_body,
        grid=(4,),
        in_specs=[
            pl.BlockSpec((tb, h), lambda tbi: (tbi, 0)),
            pl.BlockSpec((tb, h), lambda tbi: (tbi, 0)),
            pl.BlockSpec((tb, h), lambda tbi: (tbi, 0)),
            pl.BlockSpec((tb, h), lambda tbi: (tbi, 0)),
            pl.BlockSpec((tb, 2), lambda tbi: (tbi, 0)),
        ],
        out_specs=pl.BlockSpec((tb, h), lambda tbi: (tbi, 0)),
        out_shape=jax.ShapeDtypeStruct((t, h), jnp.float32),
        compiler_params=pltpu.CompilerParams(
            dimension_semantics=("arbitrary",)),
    )(shared_a, shared_b, g0, g1, scores)

    return out.reshape(b, s_, h)


# speedup vs baseline: 1.0654x; 1.0654x over previous
import functools

import jax
import jax.numpy as jnp
from jax import lax
from jax.experimental import pallas as pl
from jax.experimental.pallas import tpu as pltpu
from jax.experimental.pallas import tpu_sc as plsc

TILE = 512
_NW = 32


def _meta_body(nroute, x_ref, wr_ref, bias_ref, scores_ref, dest_ref, te_ref):
    logits = jnp.dot(x_ref[...], wr_ref[...]) * bias_ref[...]
    p = jax.nn.sigmoid(logits)
    t, e = p.shape
    ii = lax.broadcasted_iota(jnp.int32, (t, e), 1)
    m1 = jnp.max(p, axis=1, keepdims=True)
    first1 = jnp.min(jnp.where(p == m1, ii, e), axis=1, keepdims=True)
    sel1 = ii == first1
    p2 = jnp.where(sel1, -1.0, p)
    m2 = jnp.max(p2, axis=1, keepdims=True)
    first2 = jnp.min(jnp.where(p2 == m2, ii, e), axis=1, keepdims=True)
    sel2 = ii == first2
    denom = m1 + m2
    scores_ref[...] = jnp.concatenate([m1 / denom, m2 / denom], axis=1)

    oh = jnp.concatenate([sel1, sel2], axis=0).astype(jnp.float32)
    pairs = 2 * t
    nblk = pairs // 128
    oh3 = oh.reshape(nblk, 128, e)
    r_i = lax.broadcasted_iota(jnp.int32, (128, 128), 0)
    c_i = lax.broadcasted_iota(jnp.int32, (128, 128), 1)
    t128 = (r_i >= c_i).astype(jnp.bfloat16)
    t128b = jnp.broadcast_to(t128[None], (nblk, 128, 128))
    within = lax.dot_general(
        t128b, oh3.astype(jnp.bfloat16), (((2,), (1,)), ((0,), (0,))),
        preferred_element_type=jnp.float32)
    sums = within[:, 127, :]
    rb = lax.broadcasted_iota(jnp.int32, (nblk, nblk), 0)
    cb = lax.broadcasted_iota(jnp.int32, (nblk, nblk), 1)
    tnb = (rb > cb).astype(jnp.bfloat16)
    offs = jnp.dot(tnb, sums.astype(jnp.bfloat16),
                   preferred_element_type=jnp.float32)

    counts = jnp.sum(oh, axis=0, keepdims=True)
    pc = jnp.ceil(counts / TILE) * TILE
    fe_r = lax.broadcasted_iota(jnp.int32, (e, e), 0)
    fe_c = lax.broadcasted_iota(jnp.int32, (e, e), 1)
    upper = (fe_r < fe_c).astype(jnp.bfloat16)
    po = jnp.dot(pc.astype(jnp.bfloat16), upper,
                 preferred_element_type=jnp.float32)

    rank_excl = within - oh3 + offs[:, None, :]
    dest3 = jnp.sum(oh3 * (rank_excl + po.reshape(1, 1, e)),
                    axis=2)
    dest_ref[...] = dest3.astype(jnp.int32)

    ends_t = jnp.transpose(po + pc)
    ntl = te_ref.shape[1]
    ti_iota = lax.broadcasted_iota(jnp.int32, (1, ntl), 1)
    tile_start = (ti_iota * TILE).astype(jnp.float32)
    raw = jnp.sum((ends_t <= tile_start).astype(jnp.int32), axis=0,
                  keepdims=True)
    del nroute
    te_ref[...] = raw


@functools.partial(jax.jit, static_argnums=(2,))
def _dispatch(xf, dest32, slots):
    t, h = xf.shape
    rows_per = t // _NW
    mesh = plsc.VectorSubcoreMesh(core_axis_name="c", subcore_axis_name="s")

    @functools.partial(
        pl.kernel, mesh=mesh,
        out_type=jax.ShapeDtypeStruct((slots, h), jnp.float32),
        scratch_types=[
            pltpu.VMEM((rows_per,), jnp.int32),
            pltpu.VMEM((rows_per,), jnp.int32),
            pltpu.VMEM((rows_per, h), jnp.float32),
            pltpu.SemaphoreType.DMA,
            pltpu.SemaphoreType.DMA,
        ],
    )
    def body(x_hbm, dest_hbm, xg_hbm, i0_v, i1_v, rows_v, sem0, sem1):
        wid = lax.axis_index("s") * 2 + lax.axis_index("c")
        base = wid * rows_per
        pltpu.sync_copy(
            dest_hbm.at[base // 128, pl.ds(lax.rem(base, 128), rows_per)],
            i0_v)
        pltpu.sync_copy(
            dest_hbm.at[(t + base) // 128,
                        pl.ds(lax.rem(t + base, 128), rows_per)],
            i1_v)
        pltpu.sync_copy(x_hbm.at[pl.ds(base, rows_per), :], rows_v)
        c0 = pltpu.async_copy(rows_v, xg_hbm.at[i0_v], sem0)
        c1 = pltpu.async_copy(rows_v, xg_hbm.at[i1_v], sem1)
        c0.wait()
        c1.wait()

    return body(xf, dest32)


def _gather2(outbuf, dest32, t):
    slots, h = outbuf.shape
    rows_per = t // _NW
    mesh = plsc.VectorSubcoreMesh(core_axis_name="c", subcore_axis_name="s")

    @functools.partial(
        pl.kernel, mesh=mesh,
        out_type=[jax.ShapeDtypeStruct((t, h), jnp.float32),
                  jax.ShapeDtypeStruct((t, h), jnp.float32)],
        scratch_types=[
            pltpu.VMEM((rows_per,), jnp.int32),
            pltpu.VMEM((rows_per,), jnp.int32),
            pltpu.VMEM((rows_per, h), jnp.float32),
            pltpu.VMEM((rows_per, h), jnp.float32),
            pltpu.SemaphoreType.DMA,
            pltpu.SemaphoreType.DMA,
        ],
    )
    def body(ob_hbm, dest_hbm, g0_hbm, g1_hbm,
             i0_v, i1_v, r0_v, r1_v, sem0, sem1):
        wid = lax.axis_index("s") * 2 + lax.axis_index("c")
        base = wid * rows_per
        pltpu.sync_copy(
            dest_hbm.at[base // 128, pl.ds(lax.rem(base, 128), rows_per)],
            i0_v)
        pltpu.sync_copy(
            dest_hbm.at[(t + base) // 128,
                        pl.ds(lax.rem(t + base, 128), rows_per)],
            i1_v)
        c0 = pltpu.async_copy(ob_hbm.at[i0_v], r0_v, sem0)
        c1 = pltpu.async_copy(ob_hbm.at[i1_v], r1_v, sem1)
        c0.wait()
        pltpu.sync_copy(r0_v, g0_hbm.at[pl.ds(base, rows_per), :])
        c1.wait()
        pltpu.sync_copy(r1_v, g1_hbm.at[pl.ds(base, rows_per), :])

    return body(outbuf, dest32)


def _expert_body(te_ref, xg_ref, wg_ref, wu_ref, wd_ref, out_ref):
    ti = pl.program_id(0)

    @pl.when(te_ref[ti] < 8)
    def _routed():
        xb = xg_ref[...].astype(jnp.bfloat16)
        wg = wg_ref[0].astype(jnp.bfloat16)
        wu = wu_ref[0].astype(jnp.bfloat16)
        wd = wd_ref[0].astype(jnp.bfloat16)
        g = jnp.dot(xb, wg, preferred_element_type=jnp.float32)
        u = jnp.dot(xb, wu, preferred_element_type=jnp.float32)
        h = ((g * jax.nn.sigmoid(g)) * u).astype(jnp.bfloat16)
        out_ref[...] = jnp.dot(h, wd, preferred_element_type=jnp.float32)


def _shared_body(x_ref, wgs_ref, wus_ref, wds_ref, out_ref):
    ic = pl.program_id(0)
    xb = x_ref[...].astype(jnp.bfloat16)
    g = jnp.dot(xb, wgs_ref[...].astype(jnp.bfloat16),
                preferred_element_type=jnp.float32)
    u = jnp.dot(xb, wus_ref[...].astype(jnp.bfloat16),
                preferred_element_type=jnp.float32)
    h = ((g * jax.nn.sigmoid(g)) * u).astype(jnp.bfloat16)
    part = jnp.dot(h, wds_ref[...].astype(jnp.bfloat16),
                   preferred_element_type=jnp.float32)

    @pl.when(ic == 0)
    def _first():
        out_ref[...] = part

    @pl.when(ic > 0)
    def _rest():
        out_ref[...] += part


def _combine_body(sh_ref, g0_ref, g1_ref, sc_ref, out_ref):
    s = sc_ref[...]
    out_ref[...] = (sh_ref[...] + s[:, 0:1] * g0_ref[...]
                    + s[:, 1:2] * g1_ref[...])


def kernel(x, W_router, routing_bias, Wg_s, Wu_s, Wd_s, Wg, Wu, Wd):
    b, s_, h = x.shape
    t = b * s_
    e = Wg.shape[0]
    i = Wg.shape[2]
    xf = x.reshape(t, h)
    ntiles = -((-2 * t) // TILE) + e
    slots = ntiles * TILE
    nb = (2 * t) // 128
    ntl = max(32, ntiles)

    scores, dest32, te = pl.pallas_call(
        functools.partial(_meta_body, ntiles),
        out_shape=[
            jax.ShapeDtypeStruct((t, 2), jnp.float32),
            jax.ShapeDtypeStruct((nb, 128), jnp.int32),
            jax.ShapeDtypeStruct((1, ntl), jnp.int32),
        ],
    )(xf, W_router, routing_bias.reshape(1, e))

    te1 = te.reshape(ntl)

    n_ic = 3 if i % 3 == 0 else 1
    iblk = i // n_ic

    xg = _dispatch(xf, dest32, slots)

    shared = pl.pallas_call(
        _shared_body,
        grid=(n_ic,),
        in_specs=[
            pl.BlockSpec((t, h), lambda ic: (0, 0)),
            pl.BlockSpec((h, iblk), lambda ic: (0, ic)),
            pl.BlockSpec((h, iblk), lambda ic: (0, ic)),
            pl.BlockSpec((iblk, h), lambda ic: (ic, 0)),
        ],
        out_specs=pl.BlockSpec((t, h), lambda ic: (0, 0)),
        out_shape=jax.ShapeDtypeStruct((t, h), jnp.float32),
        compiler_params=pltpu.CompilerParams(
            dimension_semantics=("arbitrary",)),
    )(xf, Wg_s, Wu_s, Wd_s)

    grid_spec = pltpu.PrefetchScalarGridSpec(
        num_scalar_prefetch=1,
        grid=(ntiles,),
        in_specs=[
            pl.BlockSpec((TILE, h), lambda ti, te_r: (ti, 0)),
            pl.BlockSpec((1, h, i),
                         lambda ti, te_r: (jnp.minimum(te_r[ti], 7), 0, 0)),
            pl.BlockSpec((1, h, i),
                         lambda ti, te_r: (jnp.minimum(te_r[ti], 7), 0, 0)),
            pl.BlockSpec((1, i, h),
                         lambda ti, te_r: (jnp.minimum(te_r[ti], 7), 0, 0)),
        ],
        out_specs=pl.BlockSpec((TILE, h), lambda ti, te_r: (ti, 0)),
    )
    outbuf = pl.pallas_call(
        _expert_body,
        grid_spec=grid_spec,
        out_shape=jax.ShapeDtypeStruct((slots, h), jnp.float32),
        compiler_params=pltpu.CompilerParams(
            dimension_semantics=("arbitrary",)),
    )(te1, xg, Wg, Wu, Wd)

    g0, g1 = _gather2(outbuf, dest32, t)

    tb = t // 4
    out = pl.pallas_call(
        _combine_body,
        grid=(4,),
        in_specs=[
            pl.BlockSpec((tb, h), lambda tbi: (tbi, 0)),
            pl.BlockSpec((tb, h), lambda tbi: (tbi, 0)),
            pl.BlockSpec((tb, h), lambda tbi: (tbi, 0)),
            pl.BlockSpec((tb, 2), lambda tbi: (tbi, 0)),
        ],
        out_specs=pl.BlockSpec((tb, h), lambda tbi: (tbi, 0)),
        out_shape=jax.ShapeDtypeStruct((t, h), jnp.float32),
        compiler_params=pltpu.CompilerParams(
            dimension_semantics=("arbitrary",)),
    )(shared, g0, g1, scores)

    return out.reshape(b, s_, h)
